# baseline (speedup 1.0000x reference)
import functools

import jax
import jax.numpy as jnp
from jax.experimental import pallas as pl

NEIGH = 75
NCLS = 10
CHUNK = 2048


def _score_kernel(q_ref, t_ref, o_ref, *, nreal):
    q = q_ref[...]
    qn = q / jnp.sqrt(jnp.sum(q * q, axis=1, keepdims=True))
    t = t_ref[...]
    tn = t / jnp.sqrt(jnp.maximum(jnp.sum(t * t, axis=1, keepdims=True), 1e-30))
    s = jax.lax.dot_general(
        qn, tn, (((1,), (1,)), ((), ())), preferred_element_type=jnp.float32
    )
    col = pl.program_id(0) * CHUNK + jax.lax.broadcasted_iota(
        jnp.int32, s.shape, 1
    )
    o_ref[...] = jnp.where(col < nreal, s, -1e30)


def kernel(queries, train_activations, train_labels):
    nq, d = queries.shape
    n, _ = train_activations.shape
    npad = ((n + CHUNK - 1) // CHUNK) * CHUNK
    tpad = jnp.pad(train_activations, ((0, npad - n), (0, 0)))
    scores = pl.pallas_call(
        functools.partial(_score_kernel, nreal=n),
        grid=(npad // CHUNK,),
        in_specs=[
            pl.BlockSpec((nq, d), lambda i: (0, 0)),
            pl.BlockSpec((CHUNK, d), lambda i: (i, 0)),
        ],
        out_specs=pl.BlockSpec((nq, CHUNK), lambda i: (0, i)),
        out_shape=jax.ShapeDtypeStruct((nq, npad), jnp.float32),
    )(queries, tpad)
    _, idx = jax.lax.top_k(scores, NEIGH)
    lab = jnp.take(train_labels, jnp.minimum(idx, n - 1), axis=0)
    hist = jnp.sum(jax.nn.one_hot(lab, NCLS, dtype=jnp.int32), axis=1)
    return (NEIGH - hist).astype(jnp.float32)
